# baseline (speedup 1.0000x reference)
import functools

import jax
import jax.numpy as jnp
from jax import lax
from jax.experimental import pallas as pl
from jax.experimental.pallas import tpu as pltpu
from jax.experimental.pallas import tpu_sc as plsc

_NC, _NS, _L = 2, 16, 16
_NW = _NC * _NS
_D = 1024
_VD = _D // _L
_TR = 16


def _sc_body(x_hbm, cb_hbm, imp_hbm, out_hbm, mask_v, cv, inb, outb,
             insem, outsem, psem):
    rows = x_hbm.shape[0]
    rpw = rows // _NW
    nt = rpw // _TR
    wid = lax.axis_index("s") * _NC + lax.axis_index("c")
    base = wid * rpw

    pltpu.async_copy(cb_hbm, cv, psem).wait()
    pltpu.async_copy(imp_hbm, mask_v, psem).wait()

    def _sig_step(j, acc):
        v = mask_v[pl.ds(j * _L, _L)]
        s = 1.0 / (1.0 + jnp.exp(-v))
        mask_v[pl.ds(j * _L, _L)] = s
        return acc + s

    acc = lax.fori_loop(0, _VD, _sig_step, jnp.zeros((_L,), jnp.float32))

    total = acc[0]
    for i in range(1, _L):
        total = total + acc[i]
    tv = jnp.full((_L,), total, jnp.float32)
    sv = 1.0 / (tv / _D + 1e-6)

    def _norm_step(j, carry):
        mask_v[pl.ds(j * _L, _L)] = mask_v[pl.ds(j * _L, _L)] * sv
        return carry

    lax.fori_loop(0, _VD, _norm_step, 0)

    c0 = cv[0]
    c1 = cv[1]
    c2 = cv[2]

    vpt = _TR * _VD

    def _in_copy(t, buf):
        return pltpu.make_async_copy(
            x_hbm.at[pl.ds(base + t * _TR, _TR), :], inb.at[buf], insem)

    def _out_copy(t, buf):
        return pltpu.make_async_copy(
            outb.at[buf], out_hbm.at[pl.ds(base + t * _TR, _TR), :], outsem)

    _in_copy(0, 0).start()
    _in_copy(1, 1).start()

    def _tile_step(t, carry):
        b = lax.rem(t, 2)
        _in_copy(t, b).wait()

        @pl.when(t >= 2)
        def _():
            _out_copy(t - 2, b).wait()

        def _row_step(r, carry2):
            for f in range(_VD):
                x = inb[b, r, pl.ds(f * _L, _L)]
                m = mask_v[pl.ds(f * _L, _L)]
                xm = x * m
                y = (xm * c2 + c1) * xm + c0
                outb[b, r, pl.ds(f * _L, _L)] = y * xm
            return carry2

        lax.fori_loop(0, _TR, _row_step, 0)

        _out_copy(t, b).start()

        @pl.when(t + 2 < nt)
        def _():
            _in_copy(t + 2, b).start()

        return carry

    lax.fori_loop(0, nt, _tile_step, 0)

    _out_copy(nt - 2, lax.rem(nt - 2, 2)).wait()
    _out_copy(nt - 1, lax.rem(nt - 1, 2)).wait()


def _sc_poly(x2, cb, imp, *, interpret=False):
    rows, d = x2.shape
    mesh = plsc.VectorSubcoreMesh(core_axis_name="c", subcore_axis_name="s")
    return pl.kernel(
        _sc_body,
        out_type=jax.ShapeDtypeStruct((rows, d), jnp.float32),
        mesh=mesh,
        scratch_types=[
            pltpu.VMEM((_D,), jnp.float32),
            pltpu.VMEM((3, _L), jnp.float32),
            pltpu.VMEM((2, _TR, _D), jnp.float32),
            pltpu.VMEM((2, _TR, _D), jnp.float32),
            pltpu.SemaphoreType.DMA,
            pltpu.SemaphoreType.DMA,
            pltpu.SemaphoreType.DMA,
        ],
        interpret=interpret,
    )(x2, cb, imp)


def kernel(x, coeffs, importance):
    B, T, D = x.shape
    x2 = x.reshape(B * T, D)
    cb = jnp.broadcast_to(coeffs[:, None], (3, _L))
    out = _sc_poly(x2, cb, importance)
    return out.reshape(B, T, D)


# speedup vs baseline: 8.1698x; 8.1698x over previous
import jax
import jax.numpy as jnp
from jax.experimental import pallas as pl
from jax.experimental.pallas import tpu as pltpu


_ROWS_PER_BLOCK = 2048


def _poly_body(imp_ref, c_ref, x_ref, o_ref):
    m = jax.nn.sigmoid(imp_ref[...])
    m = m / (jnp.mean(m) + 1e-6)
    c0, c1, c2 = c_ref[0], c_ref[1], c_ref[2]
    a = c0 * m
    b = c1 * (m * m)
    g = c2 * (m * m * m)
    x = x_ref[...]
    o_ref[...] = x * (a + x * (b + x * g))


def kernel(x, coeffs, importance):
    B, T, D = x.shape
    rows = B * T
    x2 = x.reshape(rows, D)
    imp2 = importance.reshape(1, D)
    r = _ROWS_PER_BLOCK
    out = pl.pallas_call(
        _poly_body,
        grid=(rows // r,),
        in_specs=[
            pl.BlockSpec((1, D), lambda i: (0, 0)),
            pl.BlockSpec(memory_space=pltpu.MemorySpace.SMEM),
            pl.BlockSpec((r, D), lambda i: (i, 0)),
        ],
        out_specs=pl.BlockSpec((r, D), lambda i: (i, 0)),
        out_shape=jax.ShapeDtypeStruct((rows, D), jnp.float32),
    )(imp2, coeffs, x2)
    return out.reshape(B, T, D)
